# baseline (speedup 1.0000x reference)
import functools

import jax
import jax.numpy as jnp
from jax import lax
from jax.experimental import pallas as pl
from jax.experimental.pallas import tpu as pltpu
from jax.experimental.pallas import tpu_sc as plsc

_N_TOK = 39
_N_POS = 50
_PAD_POS = 56
_D = 32
_N_X = 16384


def _prep_body(x_ref, e_ref, pe_ref, idx_ref, tbl_ref):
    @pl.when(pl.program_id(0) == 0)
    def _():
        tbl_ref[...] = e_ref[...][:, None, :] + pe_ref[...]

    pos = lax.broadcasted_iota(jnp.int32, idx_ref.shape, 1)
    idx = x_ref[...][:, : _N_POS] * _N_POS + pos[:, : _N_POS]
    pad = jnp.zeros((x_ref.shape[0], _PAD_POS - _N_POS), jnp.int32)
    idx_ref[...] = jnp.concatenate([idx, pad], axis=1)


def _sc_gather(tbl, idx56):
    info = plsc.get_sparse_core_info()
    nw = info.num_cores * info.num_subcores
    rows_w = _N_X // nw
    bx = 8
    n_pairs = rows_w // (2 * bx)

    mesh = plsc.VectorSubcoreMesh(core_axis_name="c", subcore_axis_name="s")

    @functools.partial(
        pl.kernel,
        mesh=mesh,
        out_type=jax.ShapeDtypeStruct((_N_X, _N_POS, _D), jnp.float32),
        compiler_params=pltpu.CompilerParams(use_tc_tiling_on_sc=False),
        scratch_types=[
            pltpu.VMEM((rows_w, _PAD_POS), jnp.int32),
            pltpu.VMEM((bx, _PAD_POS, _D), jnp.float32),
            pltpu.VMEM((bx, _PAD_POS, _D), jnp.float32),
            pltpu.SemaphoreType.DMA,
            pltpu.SemaphoreType.DMA,
            pltpu.SemaphoreType.DMA,
            pltpu.SemaphoreType.DMA,
        ],
    )
    def k(tbl_hbm, idx_hbm, out_hbm, idx_v, buf_a, buf_b, gsem_a, gsem_b,
          ssem_a, ssem_b):
        wid = lax.axis_index("s") * info.num_cores + lax.axis_index("c")
        row0 = wid * rows_w
        pltpu.sync_copy(idx_hbm.at[pl.ds(row0, rows_w)], idx_v)

        def body(t, _):
            l0 = t * 2 * bx
            d_a = [
                pltpu.async_copy(tbl_hbm.at[idx_v.at[l0 + j]],
                                 buf_a.at[j], gsem_a)
                for j in range(bx)
            ]
            for d in d_a:
                d.wait()
            s_a = pltpu.async_copy(
                buf_a.at[:, pl.ds(0, _N_POS), :],
                out_hbm.at[pl.ds(row0 + l0, bx)], ssem_a)
            d_b = [
                pltpu.async_copy(tbl_hbm.at[idx_v.at[l0 + bx + j]],
                                 buf_b.at[j], gsem_b)
                for j in range(bx)
            ]
            for d in d_b:
                d.wait()
            s_b = pltpu.async_copy(
                buf_b.at[:, pl.ds(0, _N_POS), :],
                out_hbm.at[pl.ds(row0 + l0 + bx, bx)], ssem_b)
            s_a.wait()
            s_b.wait()
            return 0

        lax.fori_loop(0, n_pairs, body, 0)

    return k(tbl, idx56)


def kernel(x, embed_weight, pe):
    x = x.astype(jnp.int32)
    idx56, tbl3 = pl.pallas_call(
        _prep_body,
        grid=(16,),
        in_specs=[
            pl.BlockSpec((_N_X // 16, _N_POS), lambda i: (i, 0)),
            pl.BlockSpec((_N_TOK, _D), lambda i: (0, 0)),
            pl.BlockSpec((1, _N_POS, _D), lambda i: (0, 0, 0)),
        ],
        out_specs=(
            pl.BlockSpec((_N_X // 16, _PAD_POS), lambda i: (i, 0)),
            pl.BlockSpec((_N_TOK, _N_POS, _D), lambda i: (0, 0, 0)),
        ),
        out_shape=(
            jax.ShapeDtypeStruct((_N_X, _PAD_POS), jnp.int32),
            jax.ShapeDtypeStruct((_N_TOK, _N_POS, _D), jnp.float32),
        ),
    )(x, embed_weight, pe)
    tbl = tbl3.reshape(_N_TOK * _N_POS, _D)
    return _sc_gather(tbl, idx56)


# speedup vs baseline: 2.1748x; 2.1748x over previous
import functools

import jax
import jax.numpy as jnp
from jax import lax
from jax.experimental import pallas as pl
from jax.experimental.pallas import tpu as pltpu
from jax.experimental.pallas import tpu_sc as plsc

_N_TOK = 39
_N_POS = 50
_D = 32
_N_X = 16384
_ROWS = _N_X * _N_POS
_PB = 5
_LANES = 16


def _prep_body(x_ref, e_ref, pe_ref, idx_ref, tbl_ref):
    @pl.when(pl.program_id(0) == 0)
    def _():
        tbl_ref[...] = e_ref[...][:, None, :] + pe_ref[...]

    pos = lax.broadcasted_iota(jnp.int32, x_ref.shape, 1)
    idx_ref[...] = x_ref[...] * _N_POS + pos


def _sc_gather_t(tbl, idx1d):
    info = plsc.get_sparse_core_info()
    nw = info.num_cores * info.num_subcores
    rows_w = _ROWS // nw
    n_units = 4 * (_N_POS // _PB)
    u_rows = _PB * 128

    mesh = plsc.VectorSubcoreMesh(core_axis_name="c", subcore_axis_name="s")

    @functools.partial(
        pl.kernel,
        mesh=mesh,
        out_type=jax.ShapeDtypeStruct((_N_POS, 4, 128, 8, 128), jnp.float32),
        compiler_params=pltpu.CompilerParams(
            use_tc_tiling_on_sc=False, needs_layout_passes=False),
        scratch_types=[
            pltpu.VMEM((rows_w,), jnp.int32),
            pltpu.VMEM((u_rows,), jnp.int32),
            pltpu.VMEM((u_rows,), jnp.int32),
            pltpu.VMEM((u_rows, _D), jnp.float32),
            pltpu.VMEM((u_rows, _D), jnp.float32),
            pltpu.VMEM((_PB, 4, 8, 128), jnp.float32),
            pltpu.VMEM((_PB, 4, 8, 128), jnp.float32),
            pltpu.SemaphoreType.DMA,
            pltpu.SemaphoreType.DMA,
            pltpu.SemaphoreType.DMA,
            pltpu.SemaphoreType.DMA,
        ],
    )
    def k(tbl_hbm, idx_hbm, out_hbm, idx_v, gl_a, gl_b, gb_a, gb_b, tb_a,
          tb_b, gsem_a, gsem_b, osem_a, osem_b):
        wid = lax.axis_index("s") * info.num_cores + lax.axis_index("c")
        pltpu.sync_copy(idx_hbm.at[pl.ds(wid * rows_w, rows_w)], idx_v)
        lane = lax.iota(jnp.int32, _LANES)
        zero = lane * 0

        def build_gl(u, gl):
            tc = u // 10
            pb = u % 10
            base_u = tc * (128 * _N_POS) + _PB * pb
            src0 = lane * _N_POS

            def bg(g, _):
                for p in range(_PB):
                    src = src0 + (base_u + g * (_LANES * _N_POS) + p)
                    v = plsc.load_gather(idx_v, [src])
                    gl[pl.ds(p * 128 + g * _LANES, _LANES)] = v
                return 0

            lax.fori_loop(0, 128 // _LANES, bg, 0)

        def fire(gl, gb, sem):
            return [
                pltpu.async_copy(tbl_hbm.at[gl.at[pl.ds(p * 128, 128)]],
                                 gb.at[pl.ds(p * 128, 128)], sem)
                for p in range(_PB)
            ]

        def transpose(gb, tb):
            def tt(ts, _):
                tr = ts // 8
                s = ts % 8
                d = tr * 8 + s
                for p in range(_PB):
                    for lg in range(128 // _LANES):
                        rows = lane + (p * 128 + lg * _LANES)
                        v = plsc.load_gather(gb, [rows, zero + d])
                        tb[p, tr, s, pl.ds(lg * _LANES, _LANES)] = v
                return 0

            lax.fori_loop(0, 32, tt, 0)

        def out_dma(u, tb, sem):
            tc = u // 10
            pb = u % 10
            return pltpu.async_copy(
                tb, out_hbm.at[pl.ds(_PB * pb, _PB), :, wid * 4 + tc], sem)

        def body(t, _):
            u_a = 2 * t
            u_b = 2 * t + 1
            build_gl(u_a, gl_a)
            d_a = fire(gl_a, gb_a, gsem_a)
            build_gl(u_b, gl_b)
            d_b = fire(gl_b, gb_b, gsem_b)
            for d in d_a:
                d.wait()
            transpose(gb_a, tb_a)
            o_a = out_dma(u_a, tb_a, osem_a)
            for d in d_b:
                d.wait()
            transpose(gb_b, tb_b)
            o_b = out_dma(u_b, tb_b, osem_b)
            o_a.wait()
            o_b.wait()
            return 0

        lax.fori_loop(0, n_units // 2, body, 0)

    return k(tbl, idx1d)


def kernel(x, embed_weight, pe):
    x = x.astype(jnp.int32)
    idx, tbl3 = pl.pallas_call(
        _prep_body,
        grid=(16,),
        in_specs=[
            pl.BlockSpec((_N_X // 16, _N_POS), lambda i: (i, 0)),
            pl.BlockSpec((_N_TOK, _D), lambda i: (0, 0)),
            pl.BlockSpec((1, _N_POS, _D), lambda i: (0, 0, 0)),
        ],
        out_specs=(
            pl.BlockSpec((_N_X // 16, _N_POS), lambda i: (i, 0)),
            pl.BlockSpec((_N_TOK, _N_POS, _D), lambda i: (0, 0, 0)),
        ),
        out_shape=(
            jax.ShapeDtypeStruct((_N_X, _N_POS), jnp.int32),
            jax.ShapeDtypeStruct((_N_TOK, _N_POS, _D), jnp.float32),
        ),
    )(x, embed_weight, pe)
    tbl = tbl3.reshape(_N_TOK * _N_POS, _D)
    out5 = _sc_gather_t(tbl, idx.reshape(_ROWS))
    return out5.transpose(2, 4, 0, 1, 3).reshape(_N_X, _N_POS, _D)


# speedup vs baseline: 2.1766x; 1.0008x over previous
import functools

import jax
import jax.numpy as jnp
from jax import lax
from jax.experimental import pallas as pl
from jax.experimental.pallas import tpu as pltpu
from jax.experimental.pallas import tpu_sc as plsc

_N_TOK = 39
_N_POS = 50
_D = 32
_N_X = 16384
_ROWS = _N_X * _N_POS
_PB = 5
_LANES = 16


def _prep_body(x_ref, e_ref, pe_ref, idx_ref, tbl_ref):
    @pl.when(pl.program_id(0) == 0)
    def _():
        tbl_ref[...] = e_ref[...][:, None, :] + pe_ref[...]

    pos = lax.broadcasted_iota(jnp.int32, x_ref.shape, 1)
    idx_ref[...] = x_ref[...] * _N_POS + pos


def _sc_gather_t(tbl, idx1d):
    info = plsc.get_sparse_core_info()
    nw = info.num_cores * info.num_subcores
    rows_w = _ROWS // nw
    n_units = 4 * (_N_POS // _PB)
    u_rows = _PB * 128

    mesh = plsc.VectorSubcoreMesh(core_axis_name="c", subcore_axis_name="s")

    @functools.partial(
        pl.kernel,
        mesh=mesh,
        out_type=jax.ShapeDtypeStruct((_N_POS, 4, 128, 8, 128), jnp.float32),
        compiler_params=pltpu.CompilerParams(
            use_tc_tiling_on_sc=False, needs_layout_passes=False),
        scratch_types=[
            pltpu.VMEM((rows_w,), jnp.int32),
            pltpu.VMEM((u_rows,), jnp.int32),
            pltpu.VMEM((u_rows,), jnp.int32),
            pltpu.VMEM((u_rows, _D), jnp.float32),
            pltpu.VMEM((u_rows, _D), jnp.float32),
            pltpu.VMEM((_PB, 4, 8, 128), jnp.float32),
            pltpu.VMEM((_PB, 4, 8, 128), jnp.float32),
            pltpu.SemaphoreType.DMA,
            pltpu.SemaphoreType.DMA,
            pltpu.SemaphoreType.DMA,
            pltpu.SemaphoreType.DMA,
        ],
    )
    def k(tbl_hbm, idx_hbm, out_hbm, idx_v, gl_a, gl_b, gb_a, gb_b, tb_a,
          tb_b, gsem_a, gsem_b, osem_a, osem_b):
        wid = lax.axis_index("s") * info.num_cores + lax.axis_index("c")
        pltpu.sync_copy(idx_hbm.at[pl.ds(wid * rows_w, rows_w)], idx_v)
        lane = lax.iota(jnp.int32, _LANES)
        zero = lane * 0

        def build_gl(u, gl):
            tc = u // 10
            pb = u % 10
            base_u = tc * (128 * _N_POS) + _PB * pb
            src0 = lane * _N_POS

            def bg(g, _):
                for p in range(_PB):
                    src = src0 + (base_u + g * (_LANES * _N_POS) + p)
                    v = plsc.load_gather(idx_v, [src])
                    gl[pl.ds(p * 128 + g * _LANES, _LANES)] = v
                return 0

            lax.fori_loop(0, 128 // _LANES, bg, 0)

        def fire(gl, gb, sem):
            return [
                pltpu.async_copy(tbl_hbm.at[gl.at[pl.ds(p * 128, 128)]],
                                 gb.at[pl.ds(p * 128, 128)], sem)
                for p in range(_PB)
            ]

        def transpose(gb, tb):
            def tt(u, _):
                p = u // 8
                lg = u % 8
                rows = lane + (p * 128 + lg * _LANES)
                for d in range(_D):
                    v = plsc.load_gather(gb, [rows, zero + d])
                    tb[p, d // 8, d % 8, pl.ds(lg * _LANES, _LANES)] = v
                return 0

            lax.fori_loop(0, _PB * (128 // _LANES), tt, 0)

        def out_dma(u, tb, sem):
            tc = u // 10
            pb = u % 10
            return pltpu.async_copy(
                tb, out_hbm.at[pl.ds(_PB * pb, _PB), :, wid * 4 + tc], sem)

        def body(t, _):
            u_a = 2 * t
            u_b = 2 * t + 1
            build_gl(u_a, gl_a)
            d_a = fire(gl_a, gb_a, gsem_a)
            build_gl(u_b, gl_b)
            d_b = fire(gl_b, gb_b, gsem_b)
            for d in d_a:
                d.wait()
            transpose(gb_a, tb_a)
            o_a = out_dma(u_a, tb_a, osem_a)
            for d in d_b:
                d.wait()
            transpose(gb_b, tb_b)
            o_b = out_dma(u_b, tb_b, osem_b)
            o_a.wait()
            o_b.wait()
            return 0

        lax.fori_loop(0, n_units // 2, body, 0)

    return k(tbl, idx1d)


def kernel(x, embed_weight, pe):
    x = x.astype(jnp.int32)
    idx, tbl3 = pl.pallas_call(
        _prep_body,
        grid=(16,),
        in_specs=[
            pl.BlockSpec((_N_X // 16, _N_POS), lambda i: (i, 0)),
            pl.BlockSpec((_N_TOK, _D), lambda i: (0, 0)),
            pl.BlockSpec((1, _N_POS, _D), lambda i: (0, 0, 0)),
        ],
        out_specs=(
            pl.BlockSpec((_N_X // 16, _N_POS), lambda i: (i, 0)),
            pl.BlockSpec((_N_TOK, _N_POS, _D), lambda i: (0, 0, 0)),
        ),
        out_shape=(
            jax.ShapeDtypeStruct((_N_X, _N_POS), jnp.int32),
            jax.ShapeDtypeStruct((_N_TOK, _N_POS, _D), jnp.float32),
        ),
    )(x, embed_weight, pe)
    tbl = tbl3.reshape(_N_TOK * _N_POS, _D)
    out5 = _sc_gather_t(tbl, idx.reshape(_ROWS))
    return out5.transpose(2, 4, 0, 1, 3).reshape(_N_X, _N_POS, _D)


# speedup vs baseline: 5.9130x; 2.7166x over previous
import functools

import jax
import jax.numpy as jnp
from jax import lax
from jax.experimental import pallas as pl
from jax.experimental.pallas import tpu as pltpu
from jax.experimental.pallas import tpu_sc as plsc

_N_TOK = 39
_N_POS = 50
_D = 32
_N_X = 16384
_ROWS = _N_X * _N_POS
_PB = 5
_LANES = 16


def _prep_body(x_ref, e_ref, pe_ref, idx_ref, tbl_ref):
    @pl.when(pl.program_id(0) == 0)
    def _():
        tbl_ref[...] = e_ref[...][:, None, :] + pe_ref[...]

    pos = lax.broadcasted_iota(jnp.int32, x_ref.shape, 1)
    idx_ref[...] = x_ref[...] * _N_POS + pos


def _sc_gather_t(tbl, idx1d):
    info = plsc.get_sparse_core_info()
    nw = info.num_cores * info.num_subcores
    rows_w = _ROWS // nw
    n_units = 4 * (_N_POS // _PB)
    u_rows = _PB * 128

    mesh = plsc.VectorSubcoreMesh(core_axis_name="c", subcore_axis_name="s")

    @functools.partial(
        pl.kernel,
        mesh=mesh,
        out_type=jax.ShapeDtypeStruct((_N_POS, 4, 128, 8, 128), jnp.float32),
        compiler_params=pltpu.CompilerParams(
            use_tc_tiling_on_sc=False, needs_layout_passes=False),
        scratch_types=[
            pltpu.VMEM((rows_w,), jnp.int32),
            pltpu.VMEM((u_rows,), jnp.int32),
            pltpu.VMEM((u_rows,), jnp.int32),
            pltpu.VMEM((u_rows, _D), jnp.float32),
            pltpu.VMEM((u_rows, _D), jnp.float32),
            pltpu.VMEM((_PB, 4, 8, 129), jnp.float32),
            pltpu.SemaphoreType.DMA,
            pltpu.SemaphoreType.DMA,
            pltpu.SemaphoreType.DMA,
            pltpu.SemaphoreType.DMA,
        ],
    )
    def k(tbl_hbm, idx_hbm, out_hbm, idx_v, gl_a, gl_b, gb_a, gb_b, tb,
          gsem_a, gsem_b, osem_a, osem_b):
        wid = lax.axis_index("s") * info.num_cores + lax.axis_index("c")
        pltpu.sync_copy(idx_hbm.at[pl.ds(wid * rows_w, rows_w)], idx_v)
        lane = lax.iota(jnp.int32, _LANES)
        zero = lane * 0
        trv_lo = lane // 8
        trv_hi = trv_lo + 2
        sv = lane % 8

        def build_gl(u, gl):
            tc = u // 10
            pb = u % 10
            base_u = tc * (128 * _N_POS) + _PB * pb
            src0 = lane * _N_POS

            def bg(g, _):
                for p in range(_PB):
                    src = src0 + (base_u + g * (_LANES * _N_POS) + p)
                    v = plsc.load_gather(idx_v, [src])
                    gl[pl.ds(p * 128 + g * _LANES, _LANES)] = v
                return 0

            lax.fori_loop(0, 128 // _LANES, bg, 0)

        def fire(gl, gb, sem):
            return [
                pltpu.async_copy(tbl_hbm.at[gl.at[pl.ds(p * 128, 128)]],
                                 gb.at[pl.ds(p * 128, 128)], sem)
                for p in range(_PB)
            ]

        def transpose(gb):
            def tt(u, _):
                p = u // 16
                l0 = (u % 16) * 8
                pv = zero + p
                for j in range(8):
                    l = l0 + j
                    row = p * 128 + l
                    lv = zero + l
                    va = gb[row, pl.ds(0, _LANES)]
                    vb = gb[row, pl.ds(_LANES, _LANES)]
                    plsc.store_scatter(tb, [pv, trv_lo, sv, lv], va)
                    plsc.store_scatter(tb, [pv, trv_hi, sv, lv], vb)
                return 0

            lax.fori_loop(0, _PB * 16, tt, 0)

        def out_dma(u, sem):
            tc = u // 10
            pb = u % 10
            return pltpu.async_copy(
                tb.at[:, :, :, pl.ds(0, 128)],
                out_hbm.at[pl.ds(_PB * pb, _PB), :, wid * 4 + tc], sem)

        def body(t, _):
            u_a = 2 * t
            u_b = 2 * t + 1
            build_gl(u_a, gl_a)
            d_a = fire(gl_a, gb_a, gsem_a)
            build_gl(u_b, gl_b)
            d_b = fire(gl_b, gb_b, gsem_b)
            for d in d_a:
                d.wait()
            transpose(gb_a)
            o_a = out_dma(u_a, osem_a)
            for d in d_b:
                d.wait()
            o_a.wait()
            transpose(gb_b)
            o_b = out_dma(u_b, osem_b)
            o_b.wait()
            return 0

        lax.fori_loop(0, n_units // 2, body, 0)

    return k(tbl, idx1d)


def kernel(x, embed_weight, pe):
    x = x.astype(jnp.int32)
    idx, tbl3 = pl.pallas_call(
        _prep_body,
        grid=(16,),
        in_specs=[
            pl.BlockSpec((_N_X // 16, _N_POS), lambda i: (i, 0)),
            pl.BlockSpec((_N_TOK, _D), lambda i: (0, 0)),
            pl.BlockSpec((1, _N_POS, _D), lambda i: (0, 0, 0)),
        ],
        out_specs=(
            pl.BlockSpec((_N_X // 16, _N_POS), lambda i: (i, 0)),
            pl.BlockSpec((_N_TOK, _N_POS, _D), lambda i: (0, 0, 0)),
        ),
        out_shape=(
            jax.ShapeDtypeStruct((_N_X, _N_POS), jnp.int32),
            jax.ShapeDtypeStruct((_N_TOK, _N_POS, _D), jnp.float32),
        ),
    )(x, embed_weight, pe)
    tbl = tbl3.reshape(_N_TOK * _N_POS, _D)
    out5 = _sc_gather_t(tbl, idx.reshape(_ROWS))
    return out5.transpose(2, 4, 0, 1, 3).reshape(_N_X, _N_POS, _D)


# speedup vs baseline: 5.9779x; 1.0110x over previous
import functools

import jax
import jax.numpy as jnp
from jax import lax
from jax.experimental import pallas as pl
from jax.experimental.pallas import tpu as pltpu
from jax.experimental.pallas import tpu_sc as plsc

_N_TOK = 39
_N_POS = 50
_D = 32
_N_X = 16384
_ROWS = _N_X * _N_POS
_PB = 5
_LANES = 16


def _prep_body(x_ref, e_ref, pe_ref, idx_ref, tbl_ref):
    @pl.when(pl.program_id(0) == 0)
    def _():
        tbl_ref[...] = e_ref[...][:, None, :] + pe_ref[...]

    pos = lax.broadcasted_iota(jnp.int32, x_ref.shape, 1)
    idx_ref[...] = x_ref[...] * _N_POS + pos


def _sc_gather_t(tbl, idx1d):
    info = plsc.get_sparse_core_info()
    nw = info.num_cores * info.num_subcores
    rows_w = _ROWS // nw
    n_units = 4 * (_N_POS // _PB)
    u_rows = _PB * 128

    mesh = plsc.VectorSubcoreMesh(core_axis_name="c", subcore_axis_name="s")

    @functools.partial(
        pl.kernel,
        mesh=mesh,
        out_type=jax.ShapeDtypeStruct((_N_POS, 4, 128, 8, 128), jnp.float32),
        compiler_params=pltpu.CompilerParams(
            use_tc_tiling_on_sc=False, needs_layout_passes=False),
        scratch_types=[
            pltpu.VMEM((rows_w,), jnp.int32),
            pltpu.VMEM((u_rows,), jnp.int32),
            pltpu.VMEM((u_rows,), jnp.int32),
            pltpu.VMEM((u_rows, _D), jnp.float32),
            pltpu.VMEM((u_rows, _D), jnp.float32),
            pltpu.VMEM((_PB, 4, 8, 129), jnp.float32),
            pltpu.SemaphoreType.DMA,
            pltpu.SemaphoreType.DMA,
            pltpu.SemaphoreType.DMA,
            pltpu.SemaphoreType.DMA,
        ],
    )
    def k(tbl_hbm, idx_hbm, out_hbm, idx_v, gl_a, gl_b, gb_a, gb_b, tb,
          gsem_a, gsem_b, osem_a, osem_b):
        wid = lax.axis_index("s") * info.num_cores + lax.axis_index("c")
        pltpu.sync_copy(idx_hbm.at[pl.ds(wid * rows_w, rows_w)], idx_v)
        lane = lax.iota(jnp.int32, _LANES)
        zero = lane * 0
        trv_lo = lane // 8
        trv_hi = trv_lo + 2
        sv = lane % 8

        def build_gl(u, gl):
            tc = u // 10
            pb = u % 10
            base_u = tc * (128 * _N_POS) + _PB * pb
            src0 = lane * _N_POS

            def bg(g, _):
                for p in range(_PB):
                    src = src0 + (base_u + g * (_LANES * _N_POS) + p)
                    v = plsc.load_gather(idx_v, [src])
                    gl[pl.ds(p * 128 + g * _LANES, _LANES)] = v
                return 0

            lax.fori_loop(0, 128 // _LANES, bg, 0)

        def fire(gl, gb, sem):
            return [
                pltpu.async_copy(tbl_hbm.at[gl.at[pl.ds(p * 128, 128)]],
                                 gb.at[pl.ds(p * 128, 128)], sem)
                for p in range(_PB)
            ]

        def transpose(gb):
            def tt(u, _):
                p = u // 16
                l0 = (u % 16) * 8
                pv = zero + p
                for j in range(8):
                    l = l0 + j
                    row = p * 128 + l
                    lv = zero + l
                    va = gb[row, pl.ds(0, _LANES)]
                    vb = gb[row, pl.ds(_LANES, _LANES)]
                    plsc.store_scatter(tb, [pv, trv_lo, sv, lv], va)
                    plsc.store_scatter(tb, [pv, trv_hi, sv, lv], vb)
                return 0

            lax.fori_loop(0, _PB * 16, tt, 0)

        def out_dma(u, sem):
            tc = u // 10
            pb = u % 10
            return pltpu.async_copy(
                tb.at[:, :, :, pl.ds(0, 128)],
                out_hbm.at[pl.ds(_PB * pb, _PB), :, wid * 4 + tc], sem)

        def drain_b():
            pltpu.make_async_copy(
                out_hbm.at[pl.ds(0, _PB), :, 0],
                tb.at[:, :, :, pl.ds(0, 128)], osem_b).wait()

        def body(t, _):
            u_a = 2 * t
            u_b = 2 * t + 1
            build_gl(u_a, gl_a)
            d_a = fire(gl_a, gb_a, gsem_a)
            build_gl(u_b, gl_b)
            d_b = fire(gl_b, gb_b, gsem_b)
            for d in d_a:
                d.wait()

            @pl.when(t > 0)
            def _():
                drain_b()

            transpose(gb_a)
            o_a = out_dma(u_a, osem_a)
            for d in d_b:
                d.wait()
            o_a.wait()
            transpose(gb_b)
            out_dma(u_b, osem_b)
            return 0

        lax.fori_loop(0, n_units // 2, body, 0)
        drain_b()

    return k(tbl, idx1d)


def kernel(x, embed_weight, pe):
    x = x.astype(jnp.int32)
    idx, tbl3 = pl.pallas_call(
        _prep_body,
        grid=(16,),
        in_specs=[
            pl.BlockSpec((_N_X // 16, _N_POS), lambda i: (i, 0)),
            pl.BlockSpec((_N_TOK, _D), lambda i: (0, 0)),
            pl.BlockSpec((1, _N_POS, _D), lambda i: (0, 0, 0)),
        ],
        out_specs=(
            pl.BlockSpec((_N_X // 16, _N_POS), lambda i: (i, 0)),
            pl.BlockSpec((_N_TOK, _N_POS, _D), lambda i: (0, 0, 0)),
        ),
        out_shape=(
            jax.ShapeDtypeStruct((_N_X, _N_POS), jnp.int32),
            jax.ShapeDtypeStruct((_N_TOK, _N_POS, _D), jnp.float32),
        ),
    )(x, embed_weight, pe)
    tbl = tbl3.reshape(_N_TOK * _N_POS, _D)
    out5 = _sc_gather_t(tbl, idx.reshape(_ROWS))
    return out5.transpose(2, 4, 0, 1, 3).reshape(_N_X, _N_POS, _D)
